# Initial kernel scaffold; baseline (speedup 1.0000x reference)
import functools

import jax
import jax.numpy as jnp
from jax import lax
from jax.experimental import pallas as pl
from jax.experimental.pallas import tpu as pltpu
from jax.experimental.pallas import tpu_sc as plsc

_N_NODES = 100000
_DIM = 128
_T = 17
_B = 4096
_NW = 32
_PER_W = _B // _NW
_NG = _PER_W // 16
_ROWS_G = 16 * _T
_CH = 68
_NCH = _ROWS_G // _CH


def _sc_body(x_hbm, idx_hbm, s_out, p_out, idx_v, rows_v, tt_v, outs_v, outp_v, sem):
    w = lax.axis_index("s") * 2 + lax.axis_index("c")
    pltpu.sync_copy(idx_hbm.at[w], idx_v)
    iota = lax.iota(jnp.int32, 16)

    def group_body(g, pos_carry):
        copies = []
        for j in range(_NCH):
            copies.append(
                pltpu.async_copy(
                    x_hbm.at[idx_v.at[_NCH * g + j]],
                    rows_v.at[pl.ds(j * _CH, _CH)],
                    sem,
                )
            )
        for cp in copies:
            cp.wait()

        def elem_body(e, carry):
            s_vec, pos_acc = carry
            base = e * _T
            xs = [rows_v[base, pl.ds(c * 16, 16)] for c in range(8)]
            acc0 = xs[0] * xs[0]
            for c in range(1, 8):
                acc0 = acc0 + xs[c] * xs[c]
            for t in range(1, _T):
                acc = rows_v[base + t, pl.ds(0, 16)] * xs[0]
                for c in range(1, 8):
                    acc = acc + rows_v[base + t, pl.ds(c * 16, 16)] * xs[c]
                tt_v[t - 1, :] = acc
            d = plsc.load_gather(tt_v, [iota, jnp.zeros((16,), jnp.int32)])
            for l in range(1, 16):
                d = d + plsc.load_gather(tt_v, [iota, jnp.full((16,), l, jnp.int32)])
            pos_acc = pos_acc + d
            e_vec = jnp.exp(d)
            dot0 = jnp.sum(acc0)
            e0 = jnp.exp(jnp.full((16,), dot0, jnp.float32))
            s_b = jnp.sum(e_vec) + jnp.sum(e0) * (1.0 / 16.0)
            s_vec = jnp.where(iota == e, s_b, s_vec)
            return s_vec, pos_acc

        s_vec, pos_carry = lax.fori_loop(
            0, 16, elem_body, (jnp.zeros((16,), jnp.float32), pos_carry)
        )
        outs_v[g, :] = s_vec
        return pos_carry

    pos = lax.fori_loop(0, _NG, group_body, jnp.zeros((16,), jnp.float32))
    outp_v[:] = pos
    pltpu.sync_copy(outs_v, s_out.at[w])
    pltpu.sync_copy(outp_v, p_out.at[w])


@jax.jit
def _sc_call(x, idx3):
    mesh = plsc.VectorSubcoreMesh(
        core_axis_name="c", subcore_axis_name="s", num_cores=2, num_subcores=16
    )
    return pl.kernel(
        _sc_body,
        out_type=(
            jax.ShapeDtypeStruct((_NW, _NG, 16), jnp.float32),
            jax.ShapeDtypeStruct((_NW, 16), jnp.float32),
        ),
        mesh=mesh,
        scratch_types=[
            pltpu.VMEM((_NG * _NCH, _CH), jnp.int32),
            pltpu.VMEM((_ROWS_G, _DIM), jnp.float32),
            pltpu.VMEM((16, 16), jnp.float32),
            pltpu.VMEM((_NG, 16), jnp.float32),
            pltpu.VMEM((16,), jnp.float32),
            pltpu.SemaphoreType.DMA,
        ],
    )(x, idx3)


def _finish_body(s_ref, p_ref, o_ref):
    val = (16.0 * jnp.sum(jnp.log(s_ref[...])) - jnp.sum(p_ref[...])) / _B
    o_ref[...] = jnp.full((1, 1), val, jnp.float32)


@jax.jit
def _finish(s, p):
    out = pl.pallas_call(
        _finish_body,
        out_shape=jax.ShapeDtypeStruct((1, 1), jnp.float32),
    )(s.reshape(32, 128), p.reshape(4, 128))
    return out[0, 0]


def kernel(X, rt_batch):
    idx3 = rt_batch.reshape(_NW, _NG * _NCH, _CH)
    s, p = _sc_call(X, idx3)
    return _finish(s, p)


# speedup vs baseline: 4.5511x; 4.5511x over previous
import functools

import jax
import jax.numpy as jnp
from jax import lax
from jax.experimental import pallas as pl
from jax.experimental.pallas import tpu as pltpu
from jax.experimental.pallas import tpu_sc as plsc

_N_NODES = 100000
_DIM = 128
_T = 17
_B = 4096
_NW = 32
_PER_W = _B // _NW
_NG = _PER_W // 16
_ROWS_G = 16 * _T
_CH = 68
_NCH = _ROWS_G // _CH


def _sc_body(x_hbm, idx_hbm, s_out, p_out, idx_v, rows_v, tt_v, outs_v, outp_v, sem):
    w = lax.axis_index("s") * 2 + lax.axis_index("c")
    pltpu.sync_copy(idx_hbm.at[w], idx_v)
    iota = lax.iota(jnp.int32, 16)

    def group_body(g, pos_carry):
        copies = []
        for j in range(_NCH):
            copies.append(
                pltpu.async_copy(
                    x_hbm.at[idx_v.at[_NCH * g + j]],
                    rows_v.at[pl.ds(j * _CH, _CH)],
                    sem,
                )
            )
        for cp in copies:
            cp.wait()

        def elem_body(e, carry):
            s_vec, pos_acc = carry
            base = e * _T
            xs = [rows_v[base, pl.ds(c * 16, 16)] for c in range(8)]
            acc0 = xs[0] * xs[0]
            for c in range(1, 8):
                acc0 = acc0 + xs[c] * xs[c]
            for t in range(1, _T):
                acc = rows_v[base + t, pl.ds(0, 16)] * xs[0]
                for c in range(1, 8):
                    acc = acc + rows_v[base + t, pl.ds(c * 16, 16)] * xs[c]
                tt_v[pl.ds((t - 1) * 16, 16)] = acc
            iota16 = iota * 16
            d = plsc.load_gather(tt_v, [iota16])
            for l in range(1, 16):
                d = d + plsc.load_gather(tt_v, [iota16 + l])
            pos_acc = pos_acc + d
            e_vec = jnp.exp(d)
            dot0 = jnp.sum(acc0)
            e0 = jnp.exp(jnp.full((16,), dot0, jnp.float32))
            s_b = jnp.sum(e_vec) + jnp.sum(e0) * (1.0 / 16.0)
            s_vec = jnp.where(iota == e, s_b, s_vec)
            return s_vec, pos_acc

        s_vec, pos_carry = lax.fori_loop(
            0, 16, elem_body, (jnp.zeros((16,), jnp.float32), pos_carry)
        )
        outs_v[g, :] = s_vec
        return pos_carry

    pos = lax.fori_loop(0, _NG, group_body, jnp.zeros((16,), jnp.float32))
    outp_v[:] = pos
    pltpu.sync_copy(outs_v, s_out.at[w])
    pltpu.sync_copy(outp_v, p_out.at[w])


@jax.jit
def _sc_call(x, idx3):
    mesh = plsc.VectorSubcoreMesh(
        core_axis_name="c", subcore_axis_name="s", num_cores=2, num_subcores=16
    )
    return pl.kernel(
        _sc_body,
        out_type=(
            jax.ShapeDtypeStruct((_NW, _NG, 16), jnp.float32),
            jax.ShapeDtypeStruct((_NW, 16), jnp.float32),
        ),
        mesh=mesh,
        scratch_types=[
            pltpu.VMEM((_NG * _NCH, _CH), jnp.int32),
            pltpu.VMEM((_ROWS_G, _DIM), jnp.float32),
            pltpu.VMEM((256,), jnp.float32),
            pltpu.VMEM((_NG, 16), jnp.float32),
            pltpu.VMEM((16,), jnp.float32),
            pltpu.SemaphoreType.DMA,
        ],
        compiler_params=pltpu.CompilerParams(needs_layout_passes=False),
    )(x, idx3)


def _finish_body(s_ref, p_ref, o_ref):
    val = (16.0 * jnp.sum(jnp.log(s_ref[...])) - jnp.sum(p_ref[...])) / _B
    o_ref[...] = jnp.full((1, 1), val, jnp.float32)


@jax.jit
def _finish(s, p):
    out = pl.pallas_call(
        _finish_body,
        out_shape=jax.ShapeDtypeStruct((1, 1), jnp.float32),
    )(s.reshape(32, 128), p.reshape(4, 128))
    return out[0, 0]


def kernel(X, rt_batch):
    idx3 = rt_batch.reshape(_NW, _NG * _NCH, _CH)
    s, p = _sc_call(X, idx3)
    return _finish(s, p)


# speedup vs baseline: 5.2692x; 1.1578x over previous
import functools

import jax
import jax.numpy as jnp
from jax import lax
from jax.experimental import pallas as pl
from jax.experimental.pallas import tpu as pltpu
from jax.experimental.pallas import tpu_sc as plsc

_N_NODES = 100000
_DIM = 128
_T = 17
_B = 4096
_NW = 32
_PER_W = _B // _NW
_NG = _PER_W // 16
_ROWS_G = 16 * _T
_CH = 68
_NCH = _ROWS_G // _CH


def _sc_body(
    x_hbm, idx_hbm, s_out, p_out,
    idx_v, rows_v, tt_v, sa_v, se_v, outs_v, outp_v, sem0, sem1,
):
    w = lax.axis_index("s") * 2 + lax.axis_index("c")
    pltpu.sync_copy(idx_hbm.at[w], idx_v)
    iota = lax.iota(jnp.int32, 16)
    iota16 = iota * 16
    sems = (sem0, sem1)

    def fire(g, buf, sem):
        for j in range(_NCH):
            pltpu.async_copy(
                x_hbm.at[idx_v.at[_NCH * g + j]],
                rows_v.at[buf].at[pl.ds(j * _CH, _CH)],
                sem,
            )

    def drain(g, buf, sem):
        for j in range(_NCH):
            pltpu.make_async_copy(
                x_hbm.at[idx_v.at[_NCH * g + j]],
                rows_v.at[buf].at[pl.ds(j * _CH, _CH)],
                sem,
            ).wait()

    fire(0, 0, sem0)

    def outer(ii, pos_carry):
        for b in range(2):
            g = 2 * ii + b
            rows = rows_v.at[b]

            @pl.when(g + 1 < _NG)
            def _():
                fire(g + 1, 1 - b, sems[1 - b])

            drain(g, b, sems[b])

            def elem_body(e, pos_acc):
                base = e * _T
                xs = [rows[base, pl.ds(c * 16, 16)] for c in range(8)]
                acc0 = xs[0] * xs[0]
                for c in range(1, 8):
                    acc0 = acc0 + xs[c] * xs[c]
                sa_v[pl.ds(e * 16, 16)] = acc0
                for t in range(1, _T):
                    acc = rows[base + t, pl.ds(0, 16)] * xs[0]
                    for c in range(1, 8):
                        acc = acc + rows[base + t, pl.ds(c * 16, 16)] * xs[c]
                    tt_v[pl.ds((t - 1) * 16, 16)] = acc
                d = plsc.load_gather(tt_v, [iota16])
                for l in range(1, 16):
                    d = d + plsc.load_gather(tt_v, [iota16 + l])
                se_v[pl.ds(e * 16, 16)] = jnp.exp(d)
                return pos_acc + d

            pos_carry = lax.fori_loop(0, 16, elem_body, pos_carry)
            d0 = plsc.load_gather(sa_v, [iota16])
            sv = plsc.load_gather(se_v, [iota16])
            for l in range(1, 16):
                d0 = d0 + plsc.load_gather(sa_v, [iota16 + l])
                sv = sv + plsc.load_gather(se_v, [iota16 + l])
            outs_v[g, :] = sv + jnp.exp(d0)
        return pos_carry

    pos = lax.fori_loop(0, _NG // 2, outer, jnp.zeros((16,), jnp.float32))
    outp_v[:] = pos
    pltpu.sync_copy(outs_v, s_out.at[w])
    pltpu.sync_copy(outp_v, p_out.at[w])


@jax.jit
def _sc_call(x, idx3):
    mesh = plsc.VectorSubcoreMesh(
        core_axis_name="c", subcore_axis_name="s", num_cores=2, num_subcores=16
    )
    return pl.kernel(
        _sc_body,
        out_type=(
            jax.ShapeDtypeStruct((_NW, _NG, 16), jnp.float32),
            jax.ShapeDtypeStruct((_NW, 16), jnp.float32),
        ),
        mesh=mesh,
        scratch_types=[
            pltpu.VMEM((_NG * _NCH, _CH), jnp.int32),
            pltpu.VMEM((2, _ROWS_G, _DIM), jnp.float32),
            pltpu.VMEM((256,), jnp.float32),
            pltpu.VMEM((256,), jnp.float32),
            pltpu.VMEM((256,), jnp.float32),
            pltpu.VMEM((_NG, 16), jnp.float32),
            pltpu.VMEM((16,), jnp.float32),
            pltpu.SemaphoreType.DMA,
            pltpu.SemaphoreType.DMA,
        ],
        compiler_params=pltpu.CompilerParams(needs_layout_passes=False),
    )(x, idx3)


def _finish_body(s_ref, p_ref, o_ref):
    val = (16.0 * jnp.sum(jnp.log(s_ref[...])) - jnp.sum(p_ref[...])) / _B
    o_ref[...] = jnp.full((1, 1), val, jnp.float32)


@jax.jit
def _finish(s, p):
    out = pl.pallas_call(
        _finish_body,
        out_shape=jax.ShapeDtypeStruct((1, 1), jnp.float32),
    )(s.reshape(32, 128), p.reshape(4, 128))
    return out[0, 0]


def kernel(X, rt_batch):
    idx3 = rt_batch.reshape(_NW, _NG * _NCH, _CH)
    s, p = _sc_call(X, idx3)
    return _finish(s, p)


# speedup vs baseline: 6.0109x; 1.1408x over previous
import functools

import jax
import jax.numpy as jnp
from jax import lax
from jax.experimental import pallas as pl
from jax.experimental.pallas import tpu as pltpu
from jax.experimental.pallas import tpu_sc as plsc

_N_NODES = 100000
_DIM = 128
_T = 17
_B = 4096
_NW = 32
_PER_W = _B // _NW
_NG = _PER_W // 16
_ROWS_G = 16 * _T
_CH = 68
_NCH = _ROWS_G // _CH


def _sc_body(
    x_hbm, idx_hbm, s_out, p_out,
    idx_v, rows_v, tt_v, sa_v, se_v, outs_v, outp_v, sem0, sem1,
):
    w = lax.axis_index("s") * 2 + lax.axis_index("c")
    pltpu.sync_copy(idx_hbm.at[w], idx_v)
    iota = lax.iota(jnp.int32, 16)
    iota17 = iota * 17
    sems = (sem0, sem1)

    def fire(g, buf, sem):
        for j in range(_NCH):
            pltpu.async_copy(
                x_hbm.at[idx_v.at[_NCH * g + j]],
                rows_v.at[buf].at[pl.ds(j * _CH, _CH)],
                sem,
            )

    def drain(g, buf, sem):
        for j in range(_NCH):
            pltpu.make_async_copy(
                x_hbm.at[idx_v.at[_NCH * g + j]],
                rows_v.at[buf].at[pl.ds(j * _CH, _CH)],
                sem,
            ).wait()

    fire(0, 0, sem0)

    def outer(ii, pos_carry):
        for b in range(2):
            g = 2 * ii + b
            rows = rows_v.at[b]

            @pl.when(g + 1 < _NG)
            def _():
                fire(g + 1, 1 - b, sems[1 - b])

            drain(g, b, sems[b])

            def elem_body(e, pos_acc):
                base = e * _T
                xs = [rows[base, pl.ds(c * 16, 16)] for c in range(8)]
                acc0a = xs[0] * xs[0]
                acc0b = xs[1] * xs[1]
                for c in range(2, 8, 2):
                    acc0a = acc0a + xs[c] * xs[c]
                    acc0b = acc0b + xs[c + 1] * xs[c + 1]
                sa_v[pl.ds(e * 17, 16)] = acc0a + acc0b
                for t in range(1, _T):
                    acca = rows[base + t, pl.ds(0, 16)] * xs[0]
                    accb = rows[base + t, pl.ds(16, 16)] * xs[1]
                    for c in range(2, 8, 2):
                        acca = acca + rows[base + t, pl.ds(c * 16, 16)] * xs[c]
                        accb = accb + rows[base + t, pl.ds((c + 1) * 16, 16)] * xs[c + 1]
                    tt_v[pl.ds((t - 1) * 17, 16)] = acca + accb
                d = plsc.load_gather(tt_v, [iota17])
                for l in range(1, 16):
                    d = d + plsc.load_gather(tt_v, [iota17 + l])
                se_v[pl.ds(e * 17, 16)] = jnp.exp(d)
                return pos_acc + d

            pos_carry = lax.fori_loop(0, 16, elem_body, pos_carry)
            d0 = plsc.load_gather(sa_v, [iota17])
            sv = plsc.load_gather(se_v, [iota17])
            for l in range(1, 16):
                d0 = d0 + plsc.load_gather(sa_v, [iota17 + l])
                sv = sv + plsc.load_gather(se_v, [iota17 + l])
            outs_v[g, :] = sv + jnp.exp(d0)
        return pos_carry

    pos = lax.fori_loop(0, _NG // 2, outer, jnp.zeros((16,), jnp.float32))
    outp_v[:] = pos
    pltpu.sync_copy(outs_v, s_out.at[w])
    pltpu.sync_copy(outp_v, p_out.at[w])


@jax.jit
def _sc_call(x, idx3):
    mesh = plsc.VectorSubcoreMesh(
        core_axis_name="c", subcore_axis_name="s", num_cores=2, num_subcores=16
    )
    return pl.kernel(
        _sc_body,
        out_type=(
            jax.ShapeDtypeStruct((_NW, _NG, 16), jnp.float32),
            jax.ShapeDtypeStruct((_NW, 16), jnp.float32),
        ),
        mesh=mesh,
        scratch_types=[
            pltpu.VMEM((_NG * _NCH, _CH), jnp.int32),
            pltpu.VMEM((2, _ROWS_G, _DIM), jnp.float32),
            pltpu.VMEM((272,), jnp.float32),
            pltpu.VMEM((272,), jnp.float32),
            pltpu.VMEM((272,), jnp.float32),
            pltpu.VMEM((_NG, 16), jnp.float32),
            pltpu.VMEM((16,), jnp.float32),
            pltpu.SemaphoreType.DMA,
            pltpu.SemaphoreType.DMA,
        ],
        compiler_params=pltpu.CompilerParams(needs_layout_passes=False),
    )(x, idx3)


def _finish_body(s_ref, p_ref, o_ref):
    val = (16.0 * jnp.sum(jnp.log(s_ref[...])) - jnp.sum(p_ref[...])) / _B
    o_ref[...] = jnp.full((1, 1), val, jnp.float32)


@jax.jit
def _finish(s, p):
    out = pl.pallas_call(
        _finish_body,
        out_shape=jax.ShapeDtypeStruct((1, 1), jnp.float32),
    )(s.reshape(32, 128), p.reshape(4, 128))
    return out[0, 0]


def kernel(X, rt_batch):
    idx3 = rt_batch.reshape(_NW, _NG * _NCH, _CH)
    s, p = _sc_call(X, idx3)
    return _finish(s, p)
